# baseline (speedup 1.0000x reference)
import jax
import jax.numpy as jnp
import numpy as np
from jax.experimental import pallas as pl
from jax.experimental.pallas import tpu as pltpu

B = 1024
NUM_NUMERIC = 13
NUM_CAT = 26
CARD = 1000
M = NUM_NUMERIC + NUM_CAT
D = 10
F = NUM_NUMERIC + NUM_CAT * CARD
H = 200
EP = 128
ED = M * D
K0 = M * M
K1 = M * H

BT1 = 128
BT2 = 128
bf16 = jnp.bfloat16


def _split_hi(v):
    bits = jax.lax.bitcast_convert_type(v, jnp.uint32)
    hi = jax.lax.bitcast_convert_type(
        bits & np.uint32(0xFFFF0000), jnp.float32)
    return hi, v - hi


def _embed_body(x_ref, wlin_ref, wcat_ref, rep_ref, wnr_ref, e_ref):
    f32 = jnp.float32

    lin = jnp.dot(x_ref[...], wlin_ref[...], preferred_element_type=f32)

    xn_rep = jnp.dot(x_ref[:, 0:NUM_NUMERIC], rep_ref[...],
                     preferred_element_type=f32,
                     precision=jax.lax.Precision.HIGHEST)
    e_ref[:, 0:NUM_NUMERIC * D] = xn_rep * wnr_ref[...]

    for f in range(NUM_CAT):
        lo = NUM_NUMERIC + f * CARD
        col = (NUM_NUMERIC + f) * D
        e_ref[:, col:col + D] = jnp.dot(
            x_ref[:, lo:lo + CARD], wcat_ref[f], preferred_element_type=f32)
    e_ref[:, ED:ED + 1] = lin


def _cin_dnn_body(e_ref, w0_ref, b0_ref, w1_ref, b1_ref, w2_ref, b2_ref,
                  clw_ref, dw0_ref, db0_ref, dw1_ref, db1_ref, dlw_ref,
                  perm_ref, r0_ref, t0_ref, r1_ref, t1_ref, cb_ref,
                  out_ref, edm_scr):
    f32 = jnp.float32
    e390 = e_ref[:, 0:ED]
    lin = e_ref[:, ED:ED + 1]

    ehi, elo = _split_hi(e390)
    edm_scr[...] = (
        jnp.dot(ehi, perm_ref[...], preferred_element_type=f32)
        + jnp.dot(elo, perm_ref[...], preferred_element_type=f32))

    def expand(vb, vr, m_ref):
        return (jnp.dot(vb, m_ref[...], preferred_element_type=f32)
                + jnp.dot(vr, m_ref[...], preferred_element_type=f32))

    def split_bf(v):
        hi, lo = _split_hi(v)
        return hi.astype(bf16), lo.astype(bf16)

    def step(d, carry):
        p0, p1, p2 = carry
        ed = edm_scr[:, pl.ds(d * EP, EP)]
        eb, er = split_bf(ed)
        z0 = expand(eb, er, r0_ref) * expand(eb, er, t0_ref)
        c1 = jnp.dot(z0, w0_ref[...], preferred_element_type=f32) + b0_ref[...]
        eex = expand(eb, er, r1_ref)
        c1b, c1r = split_bf(c1)
        c2 = jnp.dot(eex * expand(c1b, c1r, t1_ref), w1_ref[...],
                     preferred_element_type=f32) + b1_ref[...]
        c2b, c2r = split_bf(c2)
        c3 = jnp.dot(eex * expand(c2b, c2r, t1_ref), w2_ref[...],
                     preferred_element_type=f32) + b2_ref[...]
        return (p0 + c1, p1 + c2, p2 + c3)

    zp = jnp.zeros((BT2, H), f32)
    p0, p1, p2 = jax.lax.fori_loop(0, D, step, (zp, zp, zp))
    pooled = jnp.concatenate([p0, p1, p2], axis=1)
    cin = jnp.dot(pooled, clw_ref[...], preferred_element_type=f32)

    h = jnp.maximum(jnp.dot(e390, dw0_ref[...], preferred_element_type=f32)
                    + db0_ref[...], 0.0)
    h = jnp.maximum(jnp.dot(h, dw1_ref[...], preferred_element_type=f32)
                    + db1_ref[...], 0.0)
    dnn = jnp.dot(h, dlw_ref[...], preferred_element_type=f32)

    out_ref[...] = jax.nn.sigmoid(lin + cin + dnn + cb_ref[0, 0])


@jax.jit
def kernel(x, w_lin, b_lin, W_num, W_cat,
           cin_w0, cin_b0, cin_w1, cin_b1, cin_w2, cin_b2,
           cin_lin_w, cin_lin_b,
           dnn_w0, dnn_b0, dnn_w1, dnn_b1, dnn_lin_w, dnn_lin_b, pred_b):
    f32 = jnp.float32

    repn = np.zeros((NUM_NUMERIC, NUM_NUMERIC * D), np.float32)
    repn[np.repeat(np.arange(NUM_NUMERIC), D),
         np.arange(NUM_NUMERIC * D)] = 1.0
    pf = np.zeros((ED, D * EP), np.float32)
    fidx = np.repeat(np.arange(M), D)
    didx = np.tile(np.arange(D), M)
    pf[np.arange(ED), didx * EP + fidx] = 1.0
    r0 = np.zeros((EP, K0), np.float32)
    t0 = np.zeros((EP, K0), np.float32)
    ii = np.repeat(np.arange(M), M)
    jj = np.tile(np.arange(M), M)
    r0[ii, np.arange(K0)] = 1.0
    t0[jj, np.arange(K0)] = 1.0
    r1 = np.zeros((EP, K1), np.float32)
    t1 = np.zeros((H, K1), np.float32)
    ii = np.repeat(np.arange(M), H)
    jj = np.tile(np.arange(H), M)
    r1[ii, np.arange(K1)] = 1.0
    t1[jj, np.arange(K1)] = 1.0

    perm = jnp.asarray(pf)
    repn_j = jnp.asarray(repn)
    r0_j = jnp.asarray(r0, bf16)
    t0_j = jnp.asarray(t0, bf16)
    r1_j = jnp.asarray(r1, bf16)
    t1_j = jnp.asarray(t1, bf16)

    wnr = W_num.reshape(1, NUM_NUMERIC * D)
    b0r = cin_b0.reshape(1, H)
    b1r = cin_b1.reshape(1, H)
    b2r = cin_b2.reshape(1, H)
    db0 = dnn_b0.reshape(1, -1)
    db1 = dnn_b1.reshape(1, -1)
    cb = (b_lin + cin_lin_b + dnn_lin_b + pred_b).reshape(1, 1)

    wspec = pl.BlockSpec(memory_space=pltpu.VMEM)

    e_all = pl.pallas_call(
        _embed_body,
        out_shape=jax.ShapeDtypeStruct((B, ED + 1), f32),
        grid=(B // BT1,),
        in_specs=[pl.BlockSpec((BT1, F), lambda i: (i, 0)),
                  wspec, wspec, wspec, wspec],
        out_specs=pl.BlockSpec((BT1, ED + 1), lambda i: (i, 0)),
        compiler_params=pltpu.CompilerParams(
            dimension_semantics=("parallel",),
            vmem_limit_bytes=60 * 1024 * 1024,
        ),
        name="xdeepfm_embed",
    )(x, w_lin, W_cat, repn_j, wnr)

    out = pl.pallas_call(
        _cin_dnn_body,
        out_shape=jax.ShapeDtypeStruct((B, 1), f32),
        grid=(B // BT2,),
        in_specs=[pl.BlockSpec((BT2, ED + 1), lambda i: (i, 0)),
                  wspec, wspec, wspec, wspec, wspec, wspec,
                  wspec,
                  wspec, wspec, wspec, wspec, wspec,
                  wspec, wspec, wspec, wspec, wspec,
                  pl.BlockSpec(memory_space=pltpu.SMEM)],
        out_specs=pl.BlockSpec((BT2, 1), lambda i: (i, 0)),
        scratch_shapes=[pltpu.VMEM((BT2, D * EP), f32)],
        compiler_params=pltpu.CompilerParams(
            dimension_semantics=("parallel",),
            vmem_limit_bytes=60 * 1024 * 1024,
        ),
        name="xdeepfm_cin_dnn",
    )(e_all, cin_w0, b0r, cin_w1, b1r, cin_w2, b2r, cin_lin_w,
      dnn_w0, db0, dnn_w1, db1, dnn_lin_w,
      perm, r0_j, t0_j, r1_j, t1_j, cb)
    return out


# speedup vs baseline: 1.0233x; 1.0233x over previous
import jax
import jax.numpy as jnp
import numpy as np
from jax.experimental import pallas as pl
from jax.experimental.pallas import tpu as pltpu

B = 1024
NUM_NUMERIC = 13
NUM_CAT = 26
CARD = 1000
M = NUM_NUMERIC + NUM_CAT
D = 10
F = NUM_NUMERIC + NUM_CAT * CARD
H = 200
EP = 128
HP = 256
ED = M * D
K0P = M * EP
K1P = M * HP
CH = 1280

BT1 = 128
BT2 = 64
RT = BT2 * D
bf16 = jnp.bfloat16


def _split_hi(v):
    bits = jax.lax.bitcast_convert_type(v, jnp.uint32)
    hi = jax.lax.bitcast_convert_type(
        bits & np.uint32(0xFFFF0000), jnp.float32)
    return hi, v - hi


def _split_bf(v):
    hi, lo = _split_hi(v)
    return hi.astype(bf16), lo.astype(bf16)


def _chunks(total):
    return [(lo, min(CH, total - lo)) for lo in range(0, total, CH)]


def _prep_body(c0_ref, c1_ref, c2_ref, w0p_ref, w1p_ref, w2p_ref):
    w0p_ref[...] = jnp.zeros_like(w0p_ref)
    w1p_ref[...] = jnp.zeros_like(w1p_ref)
    w2p_ref[...] = jnp.zeros_like(w2p_ref)
    for i in range(M):
        w0p_ref[i * EP:i * EP + M, :] = c0_ref[i * M:(i + 1) * M, :]
        w1p_ref[i * HP:i * HP + H, :] = c1_ref[i * H:(i + 1) * H, :]
        w2p_ref[i * HP:i * HP + H, :] = c2_ref[i * H:(i + 1) * H, :]


def _embed_body(x_ref, wlin_ref, wcat_ref, rep_ref, wnr_ref, e_ref):
    f32 = jnp.float32

    lin = jnp.dot(x_ref[...], wlin_ref[...], preferred_element_type=f32)

    xn_rep = jnp.dot(x_ref[:, 0:NUM_NUMERIC], rep_ref[...],
                     preferred_element_type=f32,
                     precision=jax.lax.Precision.HIGHEST)
    e_ref[:, 0:NUM_NUMERIC * D] = xn_rep * wnr_ref[...]

    for f in range(NUM_CAT):
        lo = NUM_NUMERIC + f * CARD
        col = (NUM_NUMERIC + f) * D
        e_ref[:, col:col + D] = jnp.dot(
            x_ref[:, lo:lo + CARD], wcat_ref[f], preferred_element_type=f32)
    e_ref[:, ED:ED + 1] = lin


def _cin_dnn_body(e_ref, w0_ref, b0_ref, w1_ref, b1_ref, w2_ref, b2_ref,
                  clw_ref, dw0_ref, db0_ref, dw1_ref, db1_ref, dlw_ref,
                  perm_ref, r0_ref, t0_ref, r1_ref, t1_ref, cb_ref,
                  out_ref, edm_scr, edst_scr):
    f32 = jnp.float32
    e390 = e_ref[:, 0:ED]
    lin = e_ref[:, ED:ED + 1]

    ehi, elo = _split_hi(e390)
    edm_scr[...] = (
        jnp.dot(ehi, perm_ref[...], preferred_element_type=f32)
        + jnp.dot(elo, perm_ref[...], preferred_element_type=f32))
    for d in range(D):
        edst_scr[d * BT2:(d + 1) * BT2, :] = edm_scr[:, d * EP:(d + 1) * EP]

    eb, er = _split_bf(edst_scr[...])

    def dot2(hi, lo, m):
        return (jnp.dot(hi, m, preferred_element_type=f32)
                + jnp.dot(lo, m, preferred_element_type=f32))

    def layer(prev, t_ref, r_ref, w_ref, b_ref, kp):
        pb, pr = _split_bf(prev)
        ctil = dot2(pb, pr, t_ref[...])
        acc = None
        for lo, w in _chunks(kp):
            eex = dot2(eb, er, r_ref[:, lo:lo + w])
            z = eex * ctil[:, 0:w]
            part = jnp.dot(z, w_ref[lo:lo + w, :], preferred_element_type=f32)
            acc = part if acc is None else acc + part
        return acc + b_ref[...]

    c1 = layer(edst_scr[...], t0_ref, r0_ref, w0_ref, b0_ref, K0P)
    c2 = layer(c1, t1_ref, r1_ref, w1_ref, b1_ref, K1P)
    c3 = layer(c2, t1_ref, r1_ref, w2_ref, b2_ref, K1P)

    p0 = c1.reshape(D, BT2, H).sum(axis=0)
    p1 = c2.reshape(D, BT2, H).sum(axis=0)
    p2 = c3.reshape(D, BT2, H).sum(axis=0)
    pooled = jnp.concatenate([p0, p1, p2], axis=1)
    cin = jnp.dot(pooled, clw_ref[...], preferred_element_type=f32)

    h = jnp.maximum(jnp.dot(e390, dw0_ref[...], preferred_element_type=f32)
                    + db0_ref[...], 0.0)
    h = jnp.maximum(jnp.dot(h, dw1_ref[...], preferred_element_type=f32)
                    + db1_ref[...], 0.0)
    dnn = jnp.dot(h, dlw_ref[...], preferred_element_type=f32)

    out_ref[...] = jax.nn.sigmoid(lin + cin + dnn + cb_ref[0, 0])


@jax.jit
def kernel(x, w_lin, b_lin, W_num, W_cat,
           cin_w0, cin_b0, cin_w1, cin_b1, cin_w2, cin_b2,
           cin_lin_w, cin_lin_b,
           dnn_w0, dnn_b0, dnn_w1, dnn_b1, dnn_lin_w, dnn_lin_b, pred_b):
    f32 = jnp.float32

    repn = np.zeros((NUM_NUMERIC, NUM_NUMERIC * D), np.float32)
    repn[np.repeat(np.arange(NUM_NUMERIC), D),
         np.arange(NUM_NUMERIC * D)] = 1.0
    pf = np.zeros((ED, D * EP), np.float32)
    fidx = np.repeat(np.arange(M), D)
    didx = np.tile(np.arange(D), M)
    pf[np.arange(ED), didx * EP + fidx] = 1.0
    r0 = np.zeros((EP, K0P), np.float32)
    r0[np.repeat(np.arange(M), EP), np.arange(K0P)] = 1.0
    r1 = np.zeros((EP, K1P), np.float32)
    r1[np.repeat(np.arange(M), HP), np.arange(K1P)] = 1.0
    t0 = np.zeros((EP, CH), np.float32)
    for i in range(CH // EP):
        t0[np.arange(M), i * EP + np.arange(M)] = 1.0
    t1 = np.zeros((H, CH), np.float32)
    for i in range(CH // HP):
        t1[np.arange(H), i * HP + np.arange(H)] = 1.0

    perm = jnp.asarray(pf)
    repn_j = jnp.asarray(repn)
    r0_j = jnp.asarray(r0, bf16)
    t0_j = jnp.asarray(t0, bf16)
    r1_j = jnp.asarray(r1, bf16)
    t1_j = jnp.asarray(t1, bf16)

    wnr = W_num.reshape(1, NUM_NUMERIC * D)
    b0r = cin_b0.reshape(1, H)
    b1r = cin_b1.reshape(1, H)
    b2r = cin_b2.reshape(1, H)
    db0 = dnn_b0.reshape(1, -1)
    db1 = dnn_b1.reshape(1, -1)
    cb = (b_lin + cin_lin_b + dnn_lin_b + pred_b).reshape(1, 1)

    wspec = pl.BlockSpec(memory_space=pltpu.VMEM)

    w0p, w1p, w2p = pl.pallas_call(
        _prep_body,
        out_shape=(jax.ShapeDtypeStruct((K0P, H), f32),
                   jax.ShapeDtypeStruct((K1P, H), f32),
                   jax.ShapeDtypeStruct((K1P, H), f32)),
        in_specs=[wspec, wspec, wspec],
        out_specs=(wspec, wspec, wspec),
        compiler_params=pltpu.CompilerParams(
            vmem_limit_bytes=60 * 1024 * 1024,
        ),
        name="xdeepfm_prep",
    )(cin_w0, cin_w1, cin_w2)

    e_all = pl.pallas_call(
        _embed_body,
        out_shape=jax.ShapeDtypeStruct((B, ED + 1), f32),
        grid=(B // BT1,),
        in_specs=[pl.BlockSpec((BT1, F), lambda i: (i, 0)),
                  wspec, wspec, wspec, wspec],
        out_specs=pl.BlockSpec((BT1, ED + 1), lambda i: (i, 0)),
        compiler_params=pltpu.CompilerParams(
            dimension_semantics=("parallel",),
            vmem_limit_bytes=60 * 1024 * 1024,
        ),
        name="xdeepfm_embed",
    )(x, w_lin, W_cat, repn_j, wnr)

    out = pl.pallas_call(
        _cin_dnn_body,
        out_shape=jax.ShapeDtypeStruct((B, 1), f32),
        grid=(B // BT2,),
        in_specs=[pl.BlockSpec((BT2, ED + 1), lambda i: (i, 0)),
                  wspec, wspec, wspec, wspec, wspec, wspec,
                  wspec,
                  wspec, wspec, wspec, wspec, wspec,
                  wspec, wspec, wspec, wspec, wspec,
                  pl.BlockSpec(memory_space=pltpu.SMEM)],
        out_specs=pl.BlockSpec((BT2, 1), lambda i: (i, 0)),
        scratch_shapes=[pltpu.VMEM((BT2, D * EP), f32),
                        pltpu.VMEM((RT, EP), f32)],
        compiler_params=pltpu.CompilerParams(
            dimension_semantics=("parallel",),
            vmem_limit_bytes=63 * 1024 * 1024,
        ),
        name="xdeepfm_cin_dnn",
    )(e_all, w0p, b0r, w1p, b1r, w2p, b2r, cin_lin_w,
      dnn_w0, db0, dnn_w1, db1, dnn_lin_w,
      perm, r0_j, t0_j, r1_j, t1_j, cb)
    return out


# speedup vs baseline: 1.2363x; 1.2081x over previous
import jax
import jax.numpy as jnp
import numpy as np
from jax.experimental import pallas as pl
from jax.experimental.pallas import tpu as pltpu

B = 1024
NUM_NUMERIC = 13
NUM_CAT = 26
CARD = 1000
M = NUM_NUMERIC + NUM_CAT
D = 10
F = NUM_NUMERIC + NUM_CAT * CARD
H = 200
EP = 128
ED = M * D
K0 = M * M
KL = M * H
CH = 3200

BT1 = 128
BT2 = 64
RT = BT2 * D
bf16 = jnp.bfloat16


def _split_hi(v):
    bits = jax.lax.bitcast_convert_type(v, jnp.uint32)
    hi = jax.lax.bitcast_convert_type(
        bits & np.uint32(0xFFFF0000), jnp.float32)
    return hi, v - hi


def _stack_bf(v):
    hi, lo = _split_hi(v)
    return jnp.concatenate([hi.astype(bf16), lo.astype(bf16)], axis=1)


def _embed_body(x_ref, wlin_ref, wcat_ref, rep_ref, wnr_ref, e_ref):
    f32 = jnp.float32

    lin = jnp.dot(x_ref[...], wlin_ref[...], preferred_element_type=f32)

    xn_rep = jnp.dot(x_ref[:, 0:NUM_NUMERIC], rep_ref[...],
                     preferred_element_type=f32,
                     precision=jax.lax.Precision.HIGHEST)
    e_ref[:, 0:NUM_NUMERIC * D] = xn_rep * wnr_ref[...]

    for f in range(NUM_CAT):
        lo = NUM_NUMERIC + f * CARD
        col = (NUM_NUMERIC + f) * D
        e_ref[:, col:col + D] = jnp.dot(
            x_ref[:, lo:lo + CARD], wcat_ref[f], preferred_element_type=f32)
    e_ref[:, ED:ED + 1] = lin


def _cin_dnn_body(e_ref, w0_ref, b0_ref, w1_ref, b1_ref, w2_ref, b2_ref,
                  clw_ref, dw0_ref, db0_ref, dw1_ref, db1_ref, dlw_ref,
                  perm_ref, r0_ref, t0_ref, r1_ref, t1_ref, cb_ref,
                  out_ref, edm_scr, edst_scr):
    f32 = jnp.float32
    e390 = e_ref[:, 0:ED]
    lin = e_ref[:, ED:ED + 1]

    ehi, elo = _split_hi(e390)
    edm_scr[...] = (
        jnp.dot(ehi, perm_ref[...], preferred_element_type=f32)
        + jnp.dot(elo, perm_ref[...], preferred_element_type=f32))
    for d in range(D):
        edst_scr[d * BT2:(d + 1) * BT2, :] = edm_scr[:, d * EP:(d + 1) * EP]

    est = _stack_bf(edst_scr[...])

    eex0 = jnp.dot(est, r0_ref[...], preferred_element_type=f32)
    ctil0 = jnp.dot(est, t0_ref[...], preferred_element_type=f32)
    c1 = jnp.dot(eex0 * ctil0, w0_ref[...],
                 preferred_element_type=f32) + b0_ref[...]

    def layer(prev, w_ref, b_ref):
        pst = _stack_bf(prev)
        ctil = jnp.dot(pst, t1_ref[...], preferred_element_type=f32)
        acc = None
        for lo in range(0, KL, CH):
            w = min(CH, KL - lo)
            eex = jnp.dot(est, r1_ref[:, lo:lo + w],
                          preferred_element_type=f32)
            part = jnp.dot(eex * ctil[:, 0:w], w_ref[lo:lo + w, :],
                           preferred_element_type=f32)
            acc = part if acc is None else acc + part
        return acc + b_ref[...]

    c2 = layer(c1, w1_ref, b1_ref)
    c3 = layer(c2, w2_ref, b2_ref)

    p0 = c1.reshape(D, BT2, H).sum(axis=0)
    p1 = c2.reshape(D, BT2, H).sum(axis=0)
    p2 = c3.reshape(D, BT2, H).sum(axis=0)
    pooled = jnp.concatenate([p0, p1, p2], axis=1)
    cin = jnp.dot(pooled, clw_ref[...], preferred_element_type=f32)

    h = jnp.maximum(jnp.dot(e390, dw0_ref[...], preferred_element_type=f32)
                    + db0_ref[...], 0.0)
    h = jnp.maximum(jnp.dot(h, dw1_ref[...], preferred_element_type=f32)
                    + db1_ref[...], 0.0)
    dnn = jnp.dot(h, dlw_ref[...], preferred_element_type=f32)

    out_ref[...] = jax.nn.sigmoid(lin + cin + dnn + cb_ref[0, 0])


@jax.jit
def kernel(x, w_lin, b_lin, W_num, W_cat,
           cin_w0, cin_b0, cin_w1, cin_b1, cin_w2, cin_b2,
           cin_lin_w, cin_lin_b,
           dnn_w0, dnn_b0, dnn_w1, dnn_b1, dnn_lin_w, dnn_lin_b, pred_b):
    f32 = jnp.float32

    repn = np.zeros((NUM_NUMERIC, NUM_NUMERIC * D), np.float32)
    repn[np.repeat(np.arange(NUM_NUMERIC), D),
         np.arange(NUM_NUMERIC * D)] = 1.0
    pf = np.zeros((ED, D * EP), np.float32)
    fidx = np.repeat(np.arange(M), D)
    didx = np.tile(np.arange(D), M)
    pf[np.arange(ED), didx * EP + fidx] = 1.0

    r0 = np.zeros((EP, K0), np.float32)
    r0[np.repeat(np.arange(M), M), np.arange(K0)] = 1.0
    t0 = np.zeros((EP, K0), np.float32)
    t0[np.tile(np.arange(M), M), np.arange(K0)] = 1.0
    r1 = np.zeros((EP, KL), np.float32)
    r1[np.repeat(np.arange(M), H), np.arange(KL)] = 1.0
    t1 = np.zeros((H, CH), np.float32)
    for i in range(CH // H):
        t1[np.arange(H), i * H + np.arange(H)] = 1.0

    perm = jnp.asarray(pf)
    repn_j = jnp.asarray(repn)
    r0_j = jnp.asarray(np.vstack([r0, r0]), bf16)
    t0_j = jnp.asarray(np.vstack([t0, t0]), bf16)
    r1_j = jnp.asarray(np.vstack([r1, r1]), bf16)
    t1_j = jnp.asarray(np.vstack([t1, t1]), bf16)

    wnr = W_num.reshape(1, NUM_NUMERIC * D)
    b0r = cin_b0.reshape(1, H)
    b1r = cin_b1.reshape(1, H)
    b2r = cin_b2.reshape(1, H)
    db0 = dnn_b0.reshape(1, -1)
    db1 = dnn_b1.reshape(1, -1)
    cb = (b_lin + cin_lin_b + dnn_lin_b + pred_b).reshape(1, 1)

    wspec = pl.BlockSpec(memory_space=pltpu.VMEM)

    e_all = pl.pallas_call(
        _embed_body,
        out_shape=jax.ShapeDtypeStruct((B, ED + 1), f32),
        grid=(B // BT1,),
        in_specs=[pl.BlockSpec((BT1, F), lambda i: (i, 0)),
                  wspec, wspec, wspec, wspec],
        out_specs=pl.BlockSpec((BT1, ED + 1), lambda i: (i, 0)),
        compiler_params=pltpu.CompilerParams(
            dimension_semantics=("parallel",),
            vmem_limit_bytes=60 * 1024 * 1024,
        ),
        name="xdeepfm_embed",
    )(x, w_lin, W_cat, repn_j, wnr)

    out = pl.pallas_call(
        _cin_dnn_body,
        out_shape=jax.ShapeDtypeStruct((B, 1), f32),
        grid=(B // BT2,),
        in_specs=[pl.BlockSpec((BT2, ED + 1), lambda i: (i, 0)),
                  wspec, wspec, wspec, wspec, wspec, wspec,
                  wspec,
                  wspec, wspec, wspec, wspec, wspec,
                  wspec, wspec, wspec, wspec, wspec,
                  pl.BlockSpec(memory_space=pltpu.SMEM)],
        out_specs=pl.BlockSpec((BT2, 1), lambda i: (i, 0)),
        scratch_shapes=[pltpu.VMEM((BT2, D * EP), f32),
                        pltpu.VMEM((RT, EP), f32)],
        compiler_params=pltpu.CompilerParams(
            dimension_semantics=("parallel",),
            vmem_limit_bytes=63 * 1024 * 1024,
        ),
        name="xdeepfm_cin_dnn",
    )(e_all, cin_w0, b0r, cin_w1, b1r, cin_w2, b2r, cin_lin_w,
      dnn_w0, db0, dnn_w1, db1, dnn_lin_w,
      perm, r0_j, t0_j, r1_j, t1_j, cb)
    return out
